# Initial kernel scaffold; baseline (speedup 1.0000x reference)
import jax
import jax.numpy as jnp
from jax.experimental import pallas as pl


def kernel(edge_index, edge_weight, user_emb, item_emb, W1, b1, W2, b2, W3, b3, g1, be1, g2, be2, fcW1, fcb1, fcW2, fcb2):
    raise NotImplementedError("write your pallas kernel here")


# speedup vs baseline: 3.4980x; 3.4980x over previous
import functools

import jax
import jax.numpy as jnp
from jax import lax
from jax.experimental import pallas as pl
from jax.experimental.pallas import tpu as pltpu
from jax.experimental.pallas import tpu_sc as plsc

_NC = 2
_NS = 16
_NW = _NC * _NS
_CHUNK = 1000


def _sc_gather(row, col, user_emb, item_emb):
    E = row.shape[0]
    D = user_emb.shape[1]
    per_w = E // _NW
    iters = per_w // _CHUNK
    mesh = plsc.VectorSubcoreMesh(
        core_axis_name="c", subcore_axis_name="s",
        num_cores=_NC, num_subcores=_NS)

    @functools.partial(
        pl.kernel,
        out_type=(jax.ShapeDtypeStruct((E, D), jnp.float32),
                  jax.ShapeDtypeStruct((E, D), jnp.float32)),
        mesh=mesh,
        scratch_types=[
            pltpu.VMEM((_CHUNK,), jnp.int32),
            pltpu.VMEM((_CHUNK,), jnp.int32),
            pltpu.VMEM((_CHUNK, D), jnp.float32),
            pltpu.VMEM((_CHUNK, D), jnp.float32),
            pltpu.SemaphoreType.DMA,
            pltpu.SemaphoreType.DMA,
        ],
        compiler_params=pltpu.CompilerParams(use_tc_tiling_on_sc=False),
    )
    def k(row_hbm, col_hbm, uemb_hbm, iemb_hbm, ue_out, ie_out,
          ridx_v, cidx_v, ue_v, ie_v, sem_u, sem_i):
        wid = lax.axis_index("s") * _NC + lax.axis_index("c")
        wbase = wid * per_w

        def body(i, carry):
            base = pl.multiple_of(wbase + i * _CHUNK, 8)
            pltpu.sync_copy(row_hbm.at[pl.ds(base, _CHUNK)], ridx_v)
            pltpu.sync_copy(col_hbm.at[pl.ds(base, _CHUNK)], cidx_v)
            cp_u = pltpu.async_copy(uemb_hbm.at[ridx_v], ue_v, sem_u)
            cp_i = pltpu.async_copy(iemb_hbm.at[cidx_v], ie_v, sem_i)
            cp_u.wait()
            cp_i.wait()
            pltpu.sync_copy(ue_v, ue_out.at[pl.ds(base, _CHUNK)])
            pltpu.sync_copy(ie_v, ie_out.at[pl.ds(base, _CHUNK)])
            return carry

        lax.fori_loop(0, iters, body, 0)

    return k(row, col, user_emb, item_emb)


def _tc_mlp(ue8, ie8, w1u, w1i, b1t, w2t, b2):
    R = ue8.shape[0]
    BLKR = 4000
    grid = R // BLKR

    def body(ue_ref, ie_ref, w1u_ref, w1i_ref, b1_ref, w2_ref, b2_ref,
             out_ref):
        h = jnp.dot(ue_ref[...], w1u_ref[...],
                    preferred_element_type=jnp.float32,
                    precision=lax.Precision.HIGHEST)
        h = h + jnp.dot(ie_ref[...], w1i_ref[...],
                        preferred_element_type=jnp.float32,
                        precision=lax.Precision.HIGHEST)
        h = jnp.maximum(h + b1_ref[...], 0.0)
        s = jnp.dot(h, w2_ref[...], preferred_element_type=jnp.float32,
                    precision=lax.Precision.HIGHEST) + b2_ref[0, 0]
        out_ref[...] = 1.0 / (1.0 + jnp.exp(-s))

    return pl.pallas_call(
        body,
        grid=(grid,),
        in_specs=[
            pl.BlockSpec((BLKR, 128), lambda i: (i, 0)),
            pl.BlockSpec((BLKR, 128), lambda i: (i, 0)),
            pl.BlockSpec((128, 512), lambda i: (0, 0)),
            pl.BlockSpec((128, 512), lambda i: (0, 0)),
            pl.BlockSpec((1, 512), lambda i: (0, 0)),
            pl.BlockSpec((512, 8), lambda i: (0, 0)),
            pl.BlockSpec((1, 1), lambda i: (0, 0)),
        ],
        out_specs=pl.BlockSpec((BLKR, 8), lambda i: (i, 0)),
        out_shape=jax.ShapeDtypeStruct((R, 8), jnp.float32),
    )(ue8, ie8, w1u, w1i, b1t, w2t, b2)


def kernel(edge_index, edge_weight, user_emb, item_emb,
           W1, b1, W2, b2, W3, b3, g1, be1, g2, be2,
           fcW1, fcb1, fcW2, fcb2):
    E = edge_index.shape[1]
    row = edge_index[0]
    col = edge_index[1]

    ue, ie = _sc_gather(row, col, user_emb, item_emb)

    eye8 = jnp.eye(8, dtype=jnp.float32)
    w1u = jnp.kron(eye8, fcW1[:16, :])
    w1i = jnp.kron(eye8, fcW1[16:, :])
    b1t = jnp.tile(fcb1, 8)[None, :]
    w2t = jnp.kron(eye8, fcW2)
    b2r = fcb2.reshape(1, 1)

    pred8 = _tc_mlp(ue.reshape(E // 8, 128), ie.reshape(E // 8, 128),
                    w1u, w1i, b1t, w2t, b2r)
    return pred8.reshape(E, 1)


# speedup vs baseline: 9.0515x; 2.5876x over previous
import functools

import jax
import jax.numpy as jnp
from jax import lax
from jax.experimental import pallas as pl
from jax.experimental.pallas import tpu as pltpu
from jax.experimental.pallas import tpu_sc as plsc

_NC = 2
_NS = 16
_NW = _NC * _NS
_CHUNK = 1000


def _sc_gather(row, col, user_emb, item_emb):
    E = row.shape[0]
    D = user_emb.shape[1]
    per_w = E // _NW
    iters = per_w // _CHUNK
    mesh = plsc.VectorSubcoreMesh(
        core_axis_name="c", subcore_axis_name="s",
        num_cores=_NC, num_subcores=_NS)

    @functools.partial(
        pl.kernel,
        out_type=(jax.ShapeDtypeStruct((E, D), jnp.float32),
                  jax.ShapeDtypeStruct((E, D), jnp.float32)),
        mesh=mesh,
        scratch_types=[
            pltpu.VMEM((_CHUNK,), jnp.int32),
            pltpu.VMEM((_CHUNK,), jnp.int32),
            pltpu.VMEM((_CHUNK, D), jnp.float32),
            pltpu.VMEM((_CHUNK, D), jnp.float32),
            pltpu.SemaphoreType.DMA,
            pltpu.SemaphoreType.DMA,
        ],
        compiler_params=pltpu.CompilerParams(use_tc_tiling_on_sc=False),
    )
    def k(row_hbm, col_hbm, uemb_hbm, iemb_hbm, ue_out, ie_out,
          ridx_v, cidx_v, ue_v, ie_v, sem_u, sem_i):
        wid = lax.axis_index("s") * _NC + lax.axis_index("c")
        wbase = wid * per_w

        def body(i, carry):
            base = pl.multiple_of(wbase + i * _CHUNK, 8)
            pltpu.sync_copy(row_hbm.at[pl.ds(base, _CHUNK)], ridx_v)
            pltpu.sync_copy(col_hbm.at[pl.ds(base, _CHUNK)], cidx_v)
            cp_u = pltpu.async_copy(uemb_hbm.at[ridx_v], ue_v, sem_u)
            cp_i = pltpu.async_copy(iemb_hbm.at[cidx_v], ie_v, sem_i)
            cp_u.wait()
            cp_i.wait()
            pltpu.sync_copy(ue_v, ue_out.at[pl.ds(base, _CHUNK)])
            pltpu.sync_copy(ie_v, ie_out.at[pl.ds(base, _CHUNK)])
            return carry

        lax.fori_loop(0, iters, body, 0)

    return k(row, col, user_emb, item_emb)


def _tc_mlp(ue8, ie8, w1u, w1i, b1t, w2t, b2):
    R = ue8.shape[0]
    BLKR = 4000
    grid = R // BLKR

    def body(ue_ref, ie_ref, w1u_ref, w1i_ref, b1_ref, w2_ref, b2_ref,
             out_ref):
        ue_b = ue_ref[...].astype(jnp.bfloat16)
        ie_b = ie_ref[...].astype(jnp.bfloat16)
        h = jnp.dot(ue_b, w1u_ref[...], preferred_element_type=jnp.float32)
        h = h + jnp.dot(ie_b, w1i_ref[...],
                        preferred_element_type=jnp.float32)
        h = jnp.maximum(h + b1_ref[...], 0.0)
        s = jnp.dot(h.astype(jnp.bfloat16), w2_ref[...],
                    preferred_element_type=jnp.float32) + b2_ref[0, 0]
        out_ref[...] = 1.0 / (1.0 + jnp.exp(-s))

    return pl.pallas_call(
        body,
        grid=(grid,),
        in_specs=[
            pl.BlockSpec((BLKR, 128), lambda i: (i, 0)),
            pl.BlockSpec((BLKR, 128), lambda i: (i, 0)),
            pl.BlockSpec((128, 512), lambda i: (0, 0)),
            pl.BlockSpec((128, 512), lambda i: (0, 0)),
            pl.BlockSpec((1, 512), lambda i: (0, 0)),
            pl.BlockSpec((512, 8), lambda i: (0, 0)),
            pl.BlockSpec((1, 1), lambda i: (0, 0)),
        ],
        out_specs=pl.BlockSpec((BLKR, 8), lambda i: (i, 0)),
        out_shape=jax.ShapeDtypeStruct((R, 8), jnp.float32),
    )(ue8, ie8, w1u, w1i, b1t, w2t, b2)


def kernel(edge_index, edge_weight, user_emb, item_emb,
           W1, b1, W2, b2, W3, b3, g1, be1, g2, be2,
           fcW1, fcb1, fcW2, fcb2):
    E = edge_index.shape[1]
    row = edge_index[0]
    col = edge_index[1]

    ue, ie = _sc_gather(row, col, user_emb, item_emb)

    eye8 = jnp.eye(8, dtype=jnp.float32)
    w1u = jnp.kron(eye8, fcW1[:16, :]).astype(jnp.bfloat16)
    w1i = jnp.kron(eye8, fcW1[16:, :]).astype(jnp.bfloat16)
    b1t = jnp.tile(fcb1, 8)[None, :]
    w2t = jnp.kron(eye8, fcW2).astype(jnp.bfloat16)
    b2r = fcb2.reshape(1, 1)

    pred8 = _tc_mlp(ue.reshape(E // 8, 128), ie.reshape(E // 8, 128),
                    w1u, w1i, b1t, w2t, b2r)
    return pred8.reshape(E, 1)


# speedup vs baseline: 9.1046x; 1.0059x over previous
import functools

import jax
import jax.numpy as jnp
from jax import lax
from jax.experimental import pallas as pl
from jax.experimental.pallas import tpu as pltpu
from jax.experimental.pallas import tpu_sc as plsc

_NC = 2
_NS = 16
_NW = _NC * _NS
_CHUNK = 1000


def _sc_gather(row, col, user_emb, item_emb):
    E = row.shape[0]
    D = user_emb.shape[1]
    per_w = E // _NW
    iters = per_w // _CHUNK
    mesh = plsc.VectorSubcoreMesh(
        core_axis_name="c", subcore_axis_name="s",
        num_cores=_NC, num_subcores=_NS)

    @functools.partial(
        pl.kernel,
        out_type=(jax.ShapeDtypeStruct((E, D), jnp.float32),
                  jax.ShapeDtypeStruct((E, D), jnp.float32)),
        mesh=mesh,
        scratch_types=[
            pltpu.VMEM((_CHUNK,), jnp.int32),
            pltpu.VMEM((_CHUNK,), jnp.int32),
            pltpu.VMEM((_CHUNK, D), jnp.float32),
            pltpu.VMEM((_CHUNK, D), jnp.float32),
            pltpu.SemaphoreType.DMA,
            pltpu.SemaphoreType.DMA,
        ],
        compiler_params=pltpu.CompilerParams(use_tc_tiling_on_sc=False),
    )
    def k(row_hbm, col_hbm, uemb_hbm, iemb_hbm, ue_out, ie_out,
          ridx_v, cidx_v, ue_v, ie_v, sem_u, sem_i):
        wid = lax.axis_index("s") * _NC + lax.axis_index("c")
        wbase = wid * per_w

        def body(i, carry):
            base = pl.multiple_of(wbase + i * _CHUNK, 8)
            pltpu.sync_copy(row_hbm.at[pl.ds(base, _CHUNK)], ridx_v)
            pltpu.sync_copy(col_hbm.at[pl.ds(base, _CHUNK)], cidx_v)
            cp_u = pltpu.async_copy(uemb_hbm.at[ridx_v], ue_v, sem_u)
            cp_i = pltpu.async_copy(iemb_hbm.at[cidx_v], ie_v, sem_i)
            cp_u.wait()
            cp_i.wait()
            pltpu.sync_copy(ue_v, ue_out.at[pl.ds(base, _CHUNK)])
            pltpu.sync_copy(ie_v, ie_out.at[pl.ds(base, _CHUNK)])
            return carry

        lax.fori_loop(0, iters, body, 0)

    return k(row, col, user_emb, item_emb)


def _tc_mlp(ue8, ie8, w1u, w1i, b1t, w2t, b2):
    R = ue8.shape[0]
    BLKR = 10000
    grid = R // BLKR

    def body(ue_ref, ie_ref, w1u_ref, w1i_ref, b1_ref, w2_ref, b2_ref,
             out_ref):
        ue_b = ue_ref[...].astype(jnp.bfloat16)
        ie_b = ie_ref[...].astype(jnp.bfloat16)
        h = jnp.dot(ue_b, w1u_ref[...], preferred_element_type=jnp.float32)
        h = h + jnp.dot(ie_b, w1i_ref[...],
                        preferred_element_type=jnp.float32)
        h = jnp.maximum(h + b1_ref[...], 0.0)
        s = jnp.dot(h.astype(jnp.bfloat16), w2_ref[...],
                    preferred_element_type=jnp.float32) + b2_ref[0, 0]
        out_ref[...] = 1.0 / (1.0 + jnp.exp(-s))

    return pl.pallas_call(
        body,
        grid=(grid,),
        in_specs=[
            pl.BlockSpec((BLKR, 128), lambda i: (i, 0)),
            pl.BlockSpec((BLKR, 128), lambda i: (i, 0)),
            pl.BlockSpec((128, 512), lambda i: (0, 0)),
            pl.BlockSpec((128, 512), lambda i: (0, 0)),
            pl.BlockSpec((1, 512), lambda i: (0, 0)),
            pl.BlockSpec((512, 8), lambda i: (0, 0)),
            pl.BlockSpec((1, 1), lambda i: (0, 0)),
        ],
        out_specs=pl.BlockSpec((BLKR, 8), lambda i: (i, 0)),
        out_shape=jax.ShapeDtypeStruct((R, 8), jnp.float32),
    )(ue8, ie8, w1u, w1i, b1t, w2t, b2)


def kernel(edge_index, edge_weight, user_emb, item_emb,
           W1, b1, W2, b2, W3, b3, g1, be1, g2, be2,
           fcW1, fcb1, fcW2, fcb2):
    E = edge_index.shape[1]
    row = edge_index[0]
    col = edge_index[1]

    ue, ie = _sc_gather(row, col, user_emb, item_emb)

    eye8 = jnp.eye(8, dtype=jnp.float32)
    w1u = jnp.kron(eye8, fcW1[:16, :]).astype(jnp.bfloat16)
    w1i = jnp.kron(eye8, fcW1[16:, :]).astype(jnp.bfloat16)
    b1t = jnp.tile(fcb1, 8)[None, :]
    w2t = jnp.kron(eye8, fcW2).astype(jnp.bfloat16)
    b2r = fcb2.reshape(1, 1)

    pred8 = _tc_mlp(ue.reshape(E // 8, 128), ie.reshape(E // 8, 128),
                    w1u, w1i, b1t, w2t, b2r)
    return pred8.reshape(E, 1)
